# baseline (speedup 1.0000x reference)
import jax
import jax.numpy as jnp
from jax import lax
from jax.experimental import pallas as pl
from jax.experimental.pallas import tpu as pltpu, tpu_sc as plsc

N = 10000
D = 128
H = 256
G = 64
GF = 32
FC1 = 196
E = 320000
E_TOT = E + N
NSUB = 16
K = 128
C = 162
PER_TILE = C * K
E_PAD = NSUB * PER_TILE
SLAB = 32
NSLAB = 8
PPC = NSLAB // 2
RT = 1000
NI = N // RT


def _tc_proj_body(acc_ref, den_ref, flag_ref, w_ref, as_ref, ad_ref,
                  h_ref, av_ref, bv_ref, mo_ref, sm_ref):
    i, j = pl.program_id(0), pl.program_id(1)
    a = acc_ref[...]
    xb = jnp.concatenate([a[q] for q in range(NSLAB)], axis=1)
    xb = xb / jnp.maximum(den_ref[...], 1e-30)
    xb = jnp.where(flag_ref[0, 0] > 0.0, jnp.maximum(xb, 0.0), xb)
    hh = jnp.dot(xb, w_ref[...], preferred_element_type=jnp.float32)
    h_ref[...] = jnp.swapaxes(hh.reshape(RT, 4, SLAB), 0, 1)
    pa = jnp.dot(hh, as_ref[...], preferred_element_type=jnp.float32)
    pb = jnp.dot(hh, ad_ref[...], preferred_element_type=jnp.float32)

    @pl.when(jnp.logical_and(i == 0, j == 0))
    def _():
        sm_ref[0] = -3e38
        sm_ref[1] = -3e38

    @pl.when(j == 0)
    def _():
        av_ref[...] = pa
        bv_ref[...] = pb

    @pl.when(j == 1)
    def _():
        av_ref[...] += pa
        bv_ref[...] += pb
        sm_ref[0] = jnp.maximum(sm_ref[0], jnp.max(av_ref[...]))
        sm_ref[1] = jnp.maximum(sm_ref[1], jnp.max(bv_ref[...]))

    @pl.when(jnp.logical_and(i == NI - 1, j == 1))
    def _():
        t = sm_ref[0] + sm_ref[1]
        mo_ref[...] = jnp.full((1, 16), jnp.where(t > 0.0, t, 0.2 * t),
                               jnp.float32)


def _tc_proj(acc, den, flag, W, a_s, a_d):
    return pl.pallas_call(
        _tc_proj_body,
        grid=(NI, 2),
        in_specs=[
            pl.BlockSpec((NSLAB, RT, SLAB), lambda i, j: (0, i, 0)),
            pl.BlockSpec((RT, 1), lambda i, j: (i, 0)),
            pl.BlockSpec((1, 1), lambda i, j: (0, 0)),
            pl.BlockSpec((H, 128), lambda i, j: (0, j)),
            pl.BlockSpec((128, 1), lambda i, j: (j, 0)),
            pl.BlockSpec((128, 1), lambda i, j: (j, 0)),
        ],
        out_specs=[
            pl.BlockSpec((4, RT, SLAB), lambda i, j: (j, i, 0)),
            pl.BlockSpec((RT, 1), lambda i, j: (i, 0)),
            pl.BlockSpec((RT, 1), lambda i, j: (i, 0)),
            pl.BlockSpec((1, 16), lambda i, j: (0, 0)),
        ],
        out_shape=[
            jax.ShapeDtypeStruct((NSLAB, N, SLAB), jnp.float32),
            jax.ShapeDtypeStruct((N, 1), jnp.float32),
            jax.ShapeDtypeStruct((N, 1), jnp.float32),
            jax.ShapeDtypeStruct((1, 16), jnp.float32),
        ],
        scratch_shapes=[pltpu.SMEM((2,), jnp.float32)],
    )(acc, den, flag, W, a_s, a_d)


def _tc_head_body(acc_ref, den_ref, b_ref, mol_ref, w1_ref, b1_ref, w2_ref,
                  b2_ref, o_ref, hg_ref, cnt_ref):
    i = pl.program_id(0)
    a = acc_ref[...]
    xb = jnp.concatenate([a[q] for q in range(NSLAB)], axis=1)
    xb = xb / jnp.maximum(den_ref[...], 1e-30)
    xb = jnp.maximum(xb, 0.0)
    gids = lax.broadcasted_iota(jnp.int32, (1, G), 1).astype(jnp.float32)
    oh = (b_ref[...] == gids).astype(jnp.float32)
    part_hg = lax.dot_general(oh, xb, (((0,), (0,)), ((), ())),
                              preferred_element_type=jnp.float32)
    ones = jnp.ones((RT, 1), jnp.float32)
    part_cnt = lax.dot_general(oh, ones, (((0,), (0,)), ((), ())),
                               preferred_element_type=jnp.float32)

    @pl.when(i == 0)
    def _():
        hg_ref[...] = part_hg
        cnt_ref[...] = part_cnt

    @pl.when(i > 0)
    def _():
        hg_ref[...] += part_hg
        cnt_ref[...] += part_cnt

    @pl.when(i == NI - 1)
    def _():
        hg = hg_ref[...] / jnp.maximum(cnt_ref[...], 1.0)
        z = jnp.concatenate([hg, mol_ref[...]], axis=1)
        z = jnp.dot(z, w1_ref[...], preferred_element_type=jnp.float32) + b1_ref[...]
        z = jnp.maximum(z, 0.0)
        o_ref[...] = jnp.dot(z, w2_ref[...], preferred_element_type=jnp.float32) + b2_ref[...]


def _tc_head(acc, den, batch_f, mol, w1, b1, w2, b2):
    return pl.pallas_call(
        _tc_head_body,
        grid=(NI,),
        in_specs=[
            pl.BlockSpec((NSLAB, RT, SLAB), lambda i: (0, i, 0)),
            pl.BlockSpec((RT, 1), lambda i: (i, 0)),
            pl.BlockSpec((RT, 1), lambda i: (i, 0)),
            pl.BlockSpec((G, GF), lambda i: (0, 0)),
            pl.BlockSpec((H + GF, FC1), lambda i: (0, 0)),
            pl.BlockSpec((1, FC1), lambda i: (0, 0)),
            pl.BlockSpec((FC1, 1), lambda i: (0, 0)),
            pl.BlockSpec((1, 1), lambda i: (0, 0)),
        ],
        out_specs=pl.BlockSpec((G, 1), lambda i: (0, 0)),
        out_shape=jax.ShapeDtypeStruct((G, 1), jnp.float32),
        scratch_shapes=[
            pltpu.VMEM((G, H), jnp.float32),
            pltpu.VMEM((G, 1), jnp.float32),
        ],
    )(acc, den, batch_f, mol, w1, b1, w2, b2)


def _sc_edge_body(h_hbm, av_hbm, bv_hbm, mo_hbm, src_hbm, dst_hbm,
                  out_hbm, den_hbm,
                  as_v, ad_v, mo_v, srcb, dstb, wb, rows0, rows1, zrows, zb,
                  accs, dens, sem0, sem1):
    cid = lax.axis_index("c")
    sid = lax.axis_index("s")

    pltpu.sync_copy(av_hbm, as_v)
    pltpu.sync_copy(bv_hbm, ad_v)
    pltpu.sync_copy(mo_hbm, mo_v)
    pltpu.sync_copy(src_hbm.at[sid], srcb)
    pltpu.sync_copy(dst_hbm.at[sid], dstb)

    z16 = jnp.zeros((16,), jnp.float32)

    def zrow(r, _):
        for q in range(SLAB // 16):
            zrows[r, pl.ds(q * 16, 16)] = z16
        return 0
    lax.fori_loop(0, K, zrow, 0)

    def zzb(r, _):
        zb[pl.ds(r * 16, 16)] = z16
        return 0
    lax.fori_loop(0, 63, zzb, 0)

    def zero_acc():
        @pl.when(sid < 10)
        def _():
            for kb in range(7):
                pltpu.sync_copy(zrows, accs.at[pl.ds(sid * 1000 + kb * K, K)])
            pltpu.sync_copy(zrows.at[pl.ds(0, 1000 - 7 * K)],
                            accs.at[pl.ds(sid * 1000 + 7 * K, 1000 - 7 * K)])

    zero_acc()

    @pl.when(sid < 10)
    def _():
        pltpu.sync_copy(zb.at[pl.ds(0, 1000)], dens.at[pl.ds(sid * 1000, 1000)])

    plsc.subcore_barrier()

    m_off = mo_v[...][0]

    ebase = sid * PER_TILE
    soff = PPC * cid * N
    lanes = lax.iota(jnp.int32, 16)

    def p1(c, _):
        for q in range(8):
            s16 = srcb[c, pl.ds(q * 16, 16)]
            d16 = dstb[c, pl.ds(q * 16, 16)]
            av = plsc.load_gather(as_v, [s16])
            bv = plsc.load_gather(ad_v, [d16])
            tt = av + bv
            e = jnp.where(tt > 0.0, tt, 0.2 * tt)
            w = jnp.exp(e - m_off)
            gid = ebase + c * K + q * 16 + lanes
            w = jnp.where(gid < E_TOT, w, 0.0)
            wb[c, pl.ds(q * 16, 16)] = w
            srcb[c, pl.ds(q * 16, 16)] = s16 + soff
        pltpu.sync_copy(wb.at[c], dens.at[dstb.at[c]], add=True)
        return 0
    with jax.named_scope("sc_p1"):
        lax.fori_loop(0, C, p1, 0)

    def scale(c, buf):
        def sgrp(g, _):
            w16 = wb[c, pl.ds(g * 16, 16)]
            for l in range(16):
                w = w16[l]
                r = g * 16 + l
                for q in range(SLAB // 16):
                    buf[r, pl.ds(q * 16, 16)] = buf[r, pl.ds(q * 16, 16)] * w
            return 0
        lax.fori_loop(0, K // 16, sgrp, 0)

    for p in range(PPC):
        pltpu.async_copy(h_hbm.at[srcb.at[0]], rows0, sem0)

        def p2(cp, _):
            c0 = cp * 2
            pltpu.async_copy(h_hbm.at[srcb.at[c0 + 1]], rows1, sem1)
            pltpu.make_async_copy(h_hbm.at[srcb.at[c0]], rows0, sem0).wait()
            scale(c0, rows0)
            pltpu.sync_copy(rows0, accs.at[dstb.at[c0]], add=True)

            @pl.when(c0 + 2 < C)
            def _():
                pltpu.async_copy(h_hbm.at[srcb.at[c0 + 2]], rows0, sem0)

            pltpu.make_async_copy(h_hbm.at[srcb.at[c0 + 1]], rows1, sem1).wait()
            scale(c0 + 1, rows1)
            pltpu.sync_copy(rows1, accs.at[dstb.at[c0 + 1]], add=True)
            return 0
        with jax.named_scope("sc_p2"):
            lax.fori_loop(0, C // 2, p2, 0)

        with jax.named_scope("sc_bar"):
            plsc.subcore_barrier()

        slab = PPC * cid + p

        @pl.when(sid < 10)
        def _():
            pltpu.sync_copy(accs.at[pl.ds(sid * 1000, 1000)],
                            out_hbm.at[pl.ds(slab * N + sid * 1000, 1000)])

        if p < PPC - 1:
            zero_acc()

            def bump(c, _):
                for q in range(8):
                    srcb[c, pl.ds(q * 16, 16)] = srcb[c, pl.ds(q * 16, 16)] + N
                return 0
            lax.fori_loop(0, C, bump, 0)
            plsc.subcore_barrier()

    @pl.when(jnp.logical_and(cid == 0, sid < 10))
    def _():
        pltpu.sync_copy(dens.at[pl.ds(sid * 1000, 1000)], zb.at[pl.ds(0, 1000)])
        pltpu.sync_copy(zb.at[pl.ds(0, 1000)], den_hbm.at[pl.ds(sid * 1000, 1000)])


_sc_edge = pl.kernel(
    _sc_edge_body,
    out_type=[
        jax.ShapeDtypeStruct((NSLAB * N, SLAB), jnp.float32),
        jax.ShapeDtypeStruct((N,), jnp.float32),
    ],
    mesh=plsc.VectorSubcoreMesh(core_axis_name="c", subcore_axis_name="s"),
    compiler_params=pltpu.CompilerParams(needs_layout_passes=False,
                                         use_tc_tiling_on_sc=False),
    scratch_types=[
        pltpu.VMEM((N,), jnp.float32),
        pltpu.VMEM((N,), jnp.float32),
        pltpu.VMEM((16,), jnp.float32),
        pltpu.VMEM((C, K), jnp.int32),
        pltpu.VMEM((C, K), jnp.int32),
        pltpu.VMEM((C, K), jnp.float32),
        pltpu.VMEM((K, SLAB), jnp.float32),
        pltpu.VMEM((K, SLAB), jnp.float32),
        pltpu.VMEM((K, SLAB), jnp.float32),
        pltpu.VMEM((1008,), jnp.float32),
        pltpu.VMEM_SHARED((N, SLAB), jnp.float32),
        pltpu.VMEM_SHARED((N,), jnp.float32),
        pltpu.SemaphoreType.DMA,
        pltpu.SemaphoreType.DMA,
    ],
)


def kernel(x, edge_index, batch, mol_feats, W1, a1s, a1d, W2, a2s, a2d,
           W3, a3s, a3d, fc1_w, fc1_b, fc2_w, fc2_b):
    loop = jnp.arange(N, dtype=jnp.int32)
    pad = jnp.zeros((E_PAD - E_TOT,), jnp.int32)
    src = jnp.concatenate([edge_index[0], loop, pad]).reshape(NSUB, C, K)
    dst = jnp.concatenate([edge_index[1], loop, pad]).reshape(NSUB, C, K)

    Ws = jnp.stack([jnp.concatenate([W1, jnp.zeros((H - D, H), jnp.float32)]),
                    W2, W3])
    As = jnp.stack([a1s, a2s, a3s]).reshape(3, H, 1)
    Ad = jnp.stack([a1d, a2d, a3d]).reshape(3, H, 1)
    flags = jnp.array([0.0, 1.0, 1.0], jnp.float32).reshape(3, 1, 1)

    xs_slabs = jnp.moveaxis(x.reshape(N, 4, SLAB), 1, 0)
    acc0 = jnp.concatenate([xs_slabs, jnp.zeros((4, N, SLAB), jnp.float32)])
    den0 = jnp.ones((N, 1), jnp.float32)

    def layer_step(carry, xs):
        acc, den = carry
        W, a_s, a_d, flag = xs
        h, av, bv, mo = _tc_proj(acc, den, flag, W, a_s, a_d)
        acc2, den2 = _sc_edge(h.reshape(NSLAB * N, SLAB), av.reshape(N),
                              bv.reshape(N), mo.reshape(16), src, dst)
        return (acc2.reshape(NSLAB, N, SLAB), den2.reshape(N, 1)), None

    (acc, den), _ = lax.scan(layer_step, (acc0, den0), (Ws, As, Ad, flags))

    out = _tc_head(acc, den, batch.astype(jnp.float32).reshape(N, 1),
                   mol_feats, fc1_w, fc1_b.reshape(1, FC1), fc2_w,
                   fc2_b.reshape(1, 1))
    return out


# speedup vs baseline: 1.0157x; 1.0157x over previous
import jax
import jax.numpy as jnp
from jax import lax
from jax.experimental import pallas as pl
from jax.experimental.pallas import tpu as pltpu, tpu_sc as plsc

N = 10000
D = 128
H = 256
G = 64
GF = 32
FC1 = 196
E = 320000
E_TOT = E + N
NSUB = 16
K = 128
C = 162
PER_TILE = C * K
E_PAD = NSUB * PER_TILE
SLAB = 32
NSLAB = 8
PPC = NSLAB // 2
RT = 1000
NI = N // RT


def _tc_proj_body(acc_ref, den_ref, flag_ref, w_ref, as_ref, ad_ref,
                  h_ref, av_ref, bv_ref, mo_ref, sm_ref):
    i, j = pl.program_id(0), pl.program_id(1)
    a = acc_ref[...]
    xb = jnp.concatenate([a[q] for q in range(NSLAB)], axis=1)
    xb = xb / jnp.maximum(den_ref[...], 1e-30)
    xb = jnp.where(flag_ref[0, 0] > 0.0, jnp.maximum(xb, 0.0), xb)
    hh = jnp.dot(xb, w_ref[...], preferred_element_type=jnp.float32)
    h_ref[...] = jnp.swapaxes(hh.reshape(RT, 4, SLAB), 0, 1)
    pa = jnp.dot(hh, as_ref[...], preferred_element_type=jnp.float32)
    pb = jnp.dot(hh, ad_ref[...], preferred_element_type=jnp.float32)

    @pl.when(jnp.logical_and(i == 0, j == 0))
    def _():
        sm_ref[0] = -3e38
        sm_ref[1] = -3e38

    @pl.when(j == 0)
    def _():
        av_ref[...] = pa
        bv_ref[...] = pb

    @pl.when(j == 1)
    def _():
        av_ref[...] += pa
        bv_ref[...] += pb
        sm_ref[0] = jnp.maximum(sm_ref[0], jnp.max(av_ref[...]))
        sm_ref[1] = jnp.maximum(sm_ref[1], jnp.max(bv_ref[...]))

    @pl.when(jnp.logical_and(i == NI - 1, j == 1))
    def _():
        t = sm_ref[0] + sm_ref[1]
        mo_ref[...] = jnp.full((1, 16), jnp.where(t > 0.0, t, 0.2 * t),
                               jnp.float32)


def _tc_proj(acc, den, flag, W, a_s, a_d):
    return pl.pallas_call(
        _tc_proj_body,
        grid=(NI, 2),
        in_specs=[
            pl.BlockSpec((NSLAB, RT, SLAB), lambda i, j: (0, i, 0)),
            pl.BlockSpec((RT, 1), lambda i, j: (i, 0)),
            pl.BlockSpec((1, 1), lambda i, j: (0, 0)),
            pl.BlockSpec((H, 128), lambda i, j: (0, j)),
            pl.BlockSpec((128, 1), lambda i, j: (j, 0)),
            pl.BlockSpec((128, 1), lambda i, j: (j, 0)),
        ],
        out_specs=[
            pl.BlockSpec((4, RT, SLAB), lambda i, j: (j, i, 0)),
            pl.BlockSpec((RT, 1), lambda i, j: (i, 0)),
            pl.BlockSpec((RT, 1), lambda i, j: (i, 0)),
            pl.BlockSpec((1, 16), lambda i, j: (0, 0)),
        ],
        out_shape=[
            jax.ShapeDtypeStruct((NSLAB, N, SLAB), jnp.float32),
            jax.ShapeDtypeStruct((N, 1), jnp.float32),
            jax.ShapeDtypeStruct((N, 1), jnp.float32),
            jax.ShapeDtypeStruct((1, 16), jnp.float32),
        ],
        scratch_shapes=[pltpu.SMEM((2,), jnp.float32)],
    )(acc, den, flag, W, a_s, a_d)


def _tc_head_body(acc_ref, den_ref, b_ref, mol_ref, w1_ref, b1_ref, w2_ref,
                  b2_ref, o_ref, hg_ref, cnt_ref):
    i = pl.program_id(0)
    a = acc_ref[...]
    xb = jnp.concatenate([a[q] for q in range(NSLAB)], axis=1)
    xb = xb / jnp.maximum(den_ref[...], 1e-30)
    xb = jnp.maximum(xb, 0.0)
    gids = lax.broadcasted_iota(jnp.int32, (1, G), 1).astype(jnp.float32)
    oh = (b_ref[...] == gids).astype(jnp.float32)
    part_hg = lax.dot_general(oh, xb, (((0,), (0,)), ((), ())),
                              preferred_element_type=jnp.float32)
    ones = jnp.ones((RT, 1), jnp.float32)
    part_cnt = lax.dot_general(oh, ones, (((0,), (0,)), ((), ())),
                               preferred_element_type=jnp.float32)

    @pl.when(i == 0)
    def _():
        hg_ref[...] = part_hg
        cnt_ref[...] = part_cnt

    @pl.when(i > 0)
    def _():
        hg_ref[...] += part_hg
        cnt_ref[...] += part_cnt

    @pl.when(i == NI - 1)
    def _():
        hg = hg_ref[...] / jnp.maximum(cnt_ref[...], 1.0)
        z = jnp.concatenate([hg, mol_ref[...]], axis=1)
        z = jnp.dot(z, w1_ref[...], preferred_element_type=jnp.float32) + b1_ref[...]
        z = jnp.maximum(z, 0.0)
        o_ref[...] = jnp.dot(z, w2_ref[...], preferred_element_type=jnp.float32) + b2_ref[...]


def _tc_head(acc, den, batch_f, mol, w1, b1, w2, b2):
    return pl.pallas_call(
        _tc_head_body,
        grid=(NI,),
        in_specs=[
            pl.BlockSpec((NSLAB, RT, SLAB), lambda i: (0, i, 0)),
            pl.BlockSpec((RT, 1), lambda i: (i, 0)),
            pl.BlockSpec((RT, 1), lambda i: (i, 0)),
            pl.BlockSpec((G, GF), lambda i: (0, 0)),
            pl.BlockSpec((H + GF, FC1), lambda i: (0, 0)),
            pl.BlockSpec((1, FC1), lambda i: (0, 0)),
            pl.BlockSpec((FC1, 1), lambda i: (0, 0)),
            pl.BlockSpec((1, 1), lambda i: (0, 0)),
        ],
        out_specs=pl.BlockSpec((G, 1), lambda i: (0, 0)),
        out_shape=jax.ShapeDtypeStruct((G, 1), jnp.float32),
        scratch_shapes=[
            pltpu.VMEM((G, H), jnp.float32),
            pltpu.VMEM((G, 1), jnp.float32),
        ],
    )(acc, den, batch_f, mol, w1, b1, w2, b2)


def _sc_edge_body(h_hbm, av_hbm, bv_hbm, mo_hbm, src_hbm, dst_hbm,
                  out_hbm, den_hbm,
                  as_v, ad_v, mo_v, srcb, dstb, wb, rows0, rows1, zrows, zb,
                  accs, dens, sem0, sem1):
    cid = lax.axis_index("c")
    sid = lax.axis_index("s")

    pltpu.sync_copy(av_hbm, as_v)
    pltpu.sync_copy(bv_hbm, ad_v)
    pltpu.sync_copy(mo_hbm, mo_v)
    pltpu.sync_copy(src_hbm.at[sid], srcb)
    pltpu.sync_copy(dst_hbm.at[sid], dstb)

    z16 = jnp.zeros((16,), jnp.float32)

    def zrow(r, _):
        for q in range(SLAB // 16):
            zrows[r, pl.ds(q * 16, 16)] = z16
        return 0
    lax.fori_loop(0, K, zrow, 0)

    def zzb(r, _):
        zb[pl.ds(r * 16, 16)] = z16
        return 0
    lax.fori_loop(0, 63, zzb, 0)

    def zero_acc():
        @pl.when(sid < 10)
        def _():
            for kb in range(7):
                pltpu.sync_copy(zrows, accs.at[pl.ds(sid * 1000 + kb * K, K)])
            pltpu.sync_copy(zrows.at[pl.ds(0, 1000 - 7 * K)],
                            accs.at[pl.ds(sid * 1000 + 7 * K, 1000 - 7 * K)])

    zero_acc()

    @pl.when(sid < 10)
    def _():
        pltpu.sync_copy(zb.at[pl.ds(0, 1000)], dens.at[pl.ds(sid * 1000, 1000)])

    plsc.subcore_barrier()

    m_off = mo_v[...][0]

    ebase = sid * PER_TILE
    soff = PPC * cid * N
    lanes = lax.iota(jnp.int32, 16)

    def p1(c, _):
        for q in range(8):
            s16 = srcb[c, pl.ds(q * 16, 16)]
            d16 = dstb[c, pl.ds(q * 16, 16)]
            av = plsc.load_gather(as_v, [s16])
            bv = plsc.load_gather(ad_v, [d16])
            tt = av + bv
            e = jnp.where(tt > 0.0, tt, 0.2 * tt)
            w = jnp.exp(e - m_off)
            gid = ebase + c * K + q * 16 + lanes
            w = jnp.where(gid < E_TOT, w, 0.0)
            wb[c, pl.ds(q * 16, 16)] = w
            srcb[c, pl.ds(q * 16, 16)] = s16 + soff
        pltpu.sync_copy(wb.at[c], dens.at[dstb.at[c]], add=True)
        return 0
    with jax.named_scope("sc_p1"):
        lax.fori_loop(0, C, p1, 0)

    def scale(c, buf):
        @plsc.parallel_loop(0, K // 16, unroll=2)
        def sgrp(g):
            w16 = wb[c, pl.ds(g * 16, 16)]
            for l in range(16):
                w = w16[l]
                r = g * 16 + l
                for q in range(SLAB // 16):
                    buf[r, pl.ds(q * 16, 16)] = buf[r, pl.ds(q * 16, 16)] * w

    for p in range(PPC):
        pltpu.async_copy(h_hbm.at[srcb.at[0]], rows0, sem0)

        def p2(cp, _):
            c0 = cp * 2
            pltpu.async_copy(h_hbm.at[srcb.at[c0 + 1]], rows1, sem1)
            pltpu.make_async_copy(h_hbm.at[srcb.at[c0]], rows0, sem0).wait()
            scale(c0, rows0)
            pltpu.sync_copy(rows0, accs.at[dstb.at[c0]], add=True)

            @pl.when(c0 + 2 < C)
            def _():
                pltpu.async_copy(h_hbm.at[srcb.at[c0 + 2]], rows0, sem0)

            pltpu.make_async_copy(h_hbm.at[srcb.at[c0 + 1]], rows1, sem1).wait()
            scale(c0 + 1, rows1)
            pltpu.sync_copy(rows1, accs.at[dstb.at[c0 + 1]], add=True)
            return 0
        with jax.named_scope("sc_p2"):
            lax.fori_loop(0, C // 2, p2, 0)

        with jax.named_scope("sc_bar"):
            plsc.subcore_barrier()

        slab = PPC * cid + p

        @pl.when(sid < 10)
        def _():
            pltpu.sync_copy(accs.at[pl.ds(sid * 1000, 1000)],
                            out_hbm.at[pl.ds(slab * N + sid * 1000, 1000)])

        if p < PPC - 1:
            zero_acc()

            def bump(c, _):
                for q in range(8):
                    srcb[c, pl.ds(q * 16, 16)] = srcb[c, pl.ds(q * 16, 16)] + N
                return 0
            lax.fori_loop(0, C, bump, 0)
            plsc.subcore_barrier()

    @pl.when(jnp.logical_and(cid == 0, sid < 10))
    def _():
        pltpu.sync_copy(dens.at[pl.ds(sid * 1000, 1000)], zb.at[pl.ds(0, 1000)])
        pltpu.sync_copy(zb.at[pl.ds(0, 1000)], den_hbm.at[pl.ds(sid * 1000, 1000)])


_sc_edge = pl.kernel(
    _sc_edge_body,
    out_type=[
        jax.ShapeDtypeStruct((NSLAB * N, SLAB), jnp.float32),
        jax.ShapeDtypeStruct((N,), jnp.float32),
    ],
    mesh=plsc.VectorSubcoreMesh(core_axis_name="c", subcore_axis_name="s"),
    compiler_params=pltpu.CompilerParams(needs_layout_passes=False,
                                         use_tc_tiling_on_sc=False),
    scratch_types=[
        pltpu.VMEM((N,), jnp.float32),
        pltpu.VMEM((N,), jnp.float32),
        pltpu.VMEM((16,), jnp.float32),
        pltpu.VMEM((C, K), jnp.int32),
        pltpu.VMEM((C, K), jnp.int32),
        pltpu.VMEM((C, K), jnp.float32),
        pltpu.VMEM((K, SLAB), jnp.float32),
        pltpu.VMEM((K, SLAB), jnp.float32),
        pltpu.VMEM((K, SLAB), jnp.float32),
        pltpu.VMEM((1008,), jnp.float32),
        pltpu.VMEM_SHARED((N, SLAB), jnp.float32),
        pltpu.VMEM_SHARED((N,), jnp.float32),
        pltpu.SemaphoreType.DMA,
        pltpu.SemaphoreType.DMA,
    ],
)


def kernel(x, edge_index, batch, mol_feats, W1, a1s, a1d, W2, a2s, a2d,
           W3, a3s, a3d, fc1_w, fc1_b, fc2_w, fc2_b):
    loop = jnp.arange(N, dtype=jnp.int32)
    pad = jnp.zeros((E_PAD - E_TOT,), jnp.int32)
    src = jnp.concatenate([edge_index[0], loop, pad]).reshape(NSUB, C, K)
    dst = jnp.concatenate([edge_index[1], loop, pad]).reshape(NSUB, C, K)

    Ws = jnp.stack([jnp.concatenate([W1, jnp.zeros((H - D, H), jnp.float32)]),
                    W2, W3])
    As = jnp.stack([a1s, a2s, a3s]).reshape(3, H, 1)
    Ad = jnp.stack([a1d, a2d, a3d]).reshape(3, H, 1)
    flags = jnp.array([0.0, 1.0, 1.0], jnp.float32).reshape(3, 1, 1)

    xs_slabs = jnp.moveaxis(x.reshape(N, 4, SLAB), 1, 0)
    acc0 = jnp.concatenate([xs_slabs, jnp.zeros((4, N, SLAB), jnp.float32)])
    den0 = jnp.ones((N, 1), jnp.float32)

    def layer_step(carry, xs):
        acc, den = carry
        W, a_s, a_d, flag = xs
        h, av, bv, mo = _tc_proj(acc, den, flag, W, a_s, a_d)
        acc2, den2 = _sc_edge(h.reshape(NSLAB * N, SLAB), av.reshape(N),
                              bv.reshape(N), mo.reshape(16), src, dst)
        return (acc2.reshape(NSLAB, N, SLAB), den2.reshape(N, 1)), None

    (acc, den), _ = lax.scan(layer_step, (acc0, den0), (Ws, As, Ad, flags))

    out = _tc_head(acc, den, batch.astype(jnp.float32).reshape(N, 1),
                   mol_feats, fc1_w, fc1_b.reshape(1, FC1), fc2_w,
                   fc2_b.reshape(1, 1))
    return out
